# Initial kernel scaffold; baseline (speedup 1.0000x reference)
import jax
import jax.numpy as jnp
from jax.experimental import pallas as pl


def kernel(hidden_states, gate_weight):
    raise NotImplementedError("write your pallas kernel here")


# speedup vs baseline: 1.1895x; 1.1895x over previous
import functools

import jax
import jax.numpy as jnp
from jax.experimental import pallas as pl
from jax.experimental.pallas import tpu as pltpu

NUM_TOKENS = 32768
HIDDEN = 4096
NUM_EXPERTS = 64
TOP_K = 8
BLOCK_T = 512


def _router_block(x_ref, w_ref, weights_ref, idx_ref):
    x = x_ref[...]
    w = w_ref[...]
    logits = jax.lax.dot_general(
        x, w, (((1,), (1,)), ((), ())), preferred_element_type=jnp.float32
    )
    m = jnp.max(logits, axis=-1, keepdims=True)
    unnorm = jnp.exp(logits - m)
    probs = unnorm / jnp.sum(unnorm, axis=-1, keepdims=True)

    cols = jax.lax.broadcasted_iota(jnp.int32, probs.shape, 1)
    work = probs
    top_w = []
    top_i = []
    for _ in range(TOP_K):
        cur = jnp.max(work, axis=-1, keepdims=True)
        hit = work == cur
        idx = jnp.min(jnp.where(hit, cols, NUM_EXPERTS), axis=-1, keepdims=True)
        top_w.append(cur)
        top_i.append(idx)
        work = jnp.where(cols == idx, -jnp.inf, work)

    weights = jnp.concatenate(top_w, axis=-1)
    weights = weights / (jnp.sum(weights, axis=-1, keepdims=True) + 1e-09)
    weights_ref[...] = weights
    idx_ref[...] = jnp.concatenate(top_i, axis=-1)


@functools.partial(jax.jit, static_argnames=())
def kernel(hidden_states, gate_weight):
    grid = (NUM_TOKENS // BLOCK_T,)
    out_shapes = (
        jax.ShapeDtypeStruct((NUM_TOKENS, TOP_K), jnp.float32),
        jax.ShapeDtypeStruct((NUM_TOKENS, TOP_K), jnp.int32),
    )
    return pl.pallas_call(
        _router_block,
        grid=grid,
        in_specs=[
            pl.BlockSpec((BLOCK_T, HIDDEN), lambda i: (i, 0)),
            pl.BlockSpec((NUM_EXPERTS, HIDDEN), lambda i: (0, 0)),
        ],
        out_specs=(
            pl.BlockSpec((BLOCK_T, TOP_K), lambda i: (i, 0)),
            pl.BlockSpec((BLOCK_T, TOP_K), lambda i: (i, 0)),
        ),
        out_shape=out_shapes,
        compiler_params=pltpu.CompilerParams(
            dimension_semantics=("arbitrary",),
        ),
    )(hidden_states, gate_weight)


# speedup vs baseline: 1.3293x; 1.1175x over previous
import functools

import jax
import jax.numpy as jnp
from jax.experimental import pallas as pl
from jax.experimental.pallas import tpu as pltpu

NUM_TOKENS = 32768
HIDDEN = 4096
NUM_EXPERTS = 64
TOP_K = 8
BLOCK_T = 512


def _router_block(x_ref, w_ref, weights_ref, idx_ref):
    x = x_ref[...]
    w = w_ref[...]
    logits = jax.lax.dot_general(
        x, w, (((1,), (1,)), ((), ())), preferred_element_type=jnp.float32
    )
    m = jnp.max(logits, axis=-1, keepdims=True)
    unnorm = jnp.exp(logits - m)
    probs = unnorm / jnp.sum(unnorm, axis=-1, keepdims=True)

    cols = jax.lax.broadcasted_iota(jnp.int32, probs.shape, 1).astype(jnp.float32)
    work = probs
    top_w = []
    top_i = []
    for _ in range(TOP_K):
        cur = jnp.max(work, axis=-1, keepdims=True)
        hit = work == cur
        idx = jnp.min(
            jnp.where(hit, cols, float(NUM_EXPERTS)), axis=-1, keepdims=True
        )
        top_w.append(cur)
        top_i.append(idx)
        work = jnp.where(cols == idx, -jnp.inf, work)

    weights = jnp.concatenate(top_w, axis=-1)
    weights = weights / (jnp.sum(weights, axis=-1, keepdims=True) + 1e-09)
    weights_ref[...] = weights
    idx_ref[...] = jnp.concatenate(top_i, axis=-1).astype(jnp.int32)


@functools.partial(jax.jit, static_argnames=())
def kernel(hidden_states, gate_weight):
    grid = (NUM_TOKENS // BLOCK_T,)
    out_shapes = (
        jax.ShapeDtypeStruct((NUM_TOKENS, TOP_K), jnp.float32),
        jax.ShapeDtypeStruct((NUM_TOKENS, TOP_K), jnp.int32),
    )
    return pl.pallas_call(
        _router_block,
        grid=grid,
        in_specs=[
            pl.BlockSpec((BLOCK_T, HIDDEN), lambda i: (i, 0)),
            pl.BlockSpec((NUM_EXPERTS, HIDDEN), lambda i: (0, 0)),
        ],
        out_specs=(
            pl.BlockSpec((BLOCK_T, TOP_K), lambda i: (i, 0)),
            pl.BlockSpec((BLOCK_T, TOP_K), lambda i: (i, 0)),
        ),
        out_shape=out_shapes,
        compiler_params=pltpu.CompilerParams(
            dimension_semantics=("arbitrary",),
        ),
    )(hidden_states, gate_weight)


# speedup vs baseline: 1.5050x; 1.1322x over previous
import functools

import jax
import jax.numpy as jnp
from jax.experimental import pallas as pl
from jax.experimental.pallas import tpu as pltpu

NUM_TOKENS = 32768
HIDDEN = 4096
NUM_EXPERTS = 64
TOP_K = 8
BLOCK_T = 1024


def _router_block(x_ref, w_ref, weights_ref, idx_ref):
    x = x_ref[...]
    w = w_ref[...]
    logits = jax.lax.dot_general(
        x, w, (((1,), (1,)), ((), ())), preferred_element_type=jnp.float32
    )
    m = jnp.max(logits, axis=-1, keepdims=True)
    unnorm = jnp.exp(logits - m)
    probs = unnorm / jnp.sum(unnorm, axis=-1, keepdims=True)

    cols = jax.lax.broadcasted_iota(jnp.int32, probs.shape, 1).astype(jnp.float32)
    work = probs
    top_w = []
    top_i = []
    for _ in range(TOP_K):
        cur = jnp.max(work, axis=-1, keepdims=True)
        hit = work == cur
        idx = jnp.min(
            jnp.where(hit, cols, float(NUM_EXPERTS)), axis=-1, keepdims=True
        )
        top_w.append(cur)
        top_i.append(idx)
        work = jnp.where(cols == idx, -jnp.inf, work)

    weights = jnp.concatenate(top_w, axis=-1)
    weights = weights / (jnp.sum(weights, axis=-1, keepdims=True) + 1e-09)
    weights_ref[...] = weights
    idx_ref[...] = jnp.concatenate(top_i, axis=-1).astype(jnp.int32)


@functools.partial(jax.jit, static_argnames=())
def kernel(hidden_states, gate_weight):
    grid = (NUM_TOKENS // BLOCK_T,)
    out_shapes = (
        jax.ShapeDtypeStruct((NUM_TOKENS, TOP_K), jnp.float32),
        jax.ShapeDtypeStruct((NUM_TOKENS, TOP_K), jnp.int32),
    )
    return pl.pallas_call(
        _router_block,
        grid=grid,
        in_specs=[
            pl.BlockSpec((BLOCK_T, HIDDEN), lambda i: (i, 0)),
            pl.BlockSpec((NUM_EXPERTS, HIDDEN), lambda i: (0, 0)),
        ],
        out_specs=(
            pl.BlockSpec((BLOCK_T, TOP_K), lambda i: (i, 0)),
            pl.BlockSpec((BLOCK_T, TOP_K), lambda i: (i, 0)),
        ),
        out_shape=out_shapes,
        compiler_params=pltpu.CompilerParams(
            dimension_semantics=("arbitrary",),
        ),
    )(hidden_states, gate_weight)
